# baseline (speedup 1.0000x reference)
import functools

import jax
import jax.numpy as jnp
from jax import lax
from jax.experimental import pallas as pl
from jax.experimental.pallas import tpu as pltpu
from jax.experimental.pallas import tpu_sc as plsc

_B, _C, _H, _W = 4, 96, 224, 224
_ROW = _H * _W
_NROWS = _B * _C

_INFO = plsc.get_sparse_core_info()
_NC, _NS = _INFO.num_cores, _INFO.num_subcores
_NWORK = _NC * _NS
_RPW = _NROWS // _NWORK


_PERM = (19, 76, 54, 90, 30, 7, 6, 35, 23)


@functools.partial(
    pl.kernel,
    mesh=plsc.VectorSubcoreMesh(core_axis_name="c", subcore_axis_name="s"),
    out_type=jax.ShapeDtypeStruct((_NROWS, _ROW), jnp.float32),
    scratch_types=[
        pltpu.VMEM((1, _ROW), jnp.float32),
        pltpu.VMEM((1, _ROW), jnp.float32),
        pltpu.SemaphoreType.DMA,
        pltpu.SemaphoreType.DMA,
        pltpu.SemaphoreType.DMA,
        pltpu.SemaphoreType.DMA,
    ],
)
def _sc_replace(x_hbm, out_hbm, buf0, buf1, g0, g1, s0, s1):
    wid = lax.axis_index("s") * _NC + lax.axis_index("c")
    r0 = wid * _RPW
    bufs = (buf0, buf1)
    gsem = (g0, g1)
    ssem = (s0, s1)

    def gather(i):
        r = r0 + i
        c = lax.rem(r, _C)
        d = jnp.int32(0)
        for p in _PERM:
            d = d + jnp.where(c == p, 1, 0).astype(jnp.int32)
        return pltpu.async_copy(
            x_hbm.at[pl.ds(r + d, 1)], bufs[i % 2], gsem[i % 2])

    def scatter(i):
        return pltpu.async_copy(
            bufs[i % 2], out_hbm.at[pl.ds(r0 + i, 1)], ssem[i % 2])

    gathers = {0: gather(0), 1: gather(1)}
    scatters = {}
    for i in range(_RPW):
        gathers[i].wait()
        scatters[i] = scatter(i)
        if i + 2 < _RPW:
            scatters[i].wait()
            gathers[i + 2] = gather(i + 2)
    scatters[_RPW - 2].wait()
    scatters[_RPW - 1].wait()


def kernel(x):
    out = _sc_replace(x.reshape(_NROWS, _ROW))
    return out.reshape(_B, _C, _H, _W)
